# Initial kernel scaffold; baseline (speedup 1.0000x reference)
import functools

import jax
import jax.numpy as jnp
from jax import lax
from jax.experimental import pallas as pl
from jax.experimental.pallas import tpu as pltpu
from jax.experimental.pallas import tpu_sc as plsc

N_NODES = 10000
N_EDGES = 320000
F_IN = 128
F_HID = 16

NC = 2
NS = 16
NW = NC * NS
NPAD = 10240
ROWS_PER_TILE = NPAD // NS
EPW = N_EDGES // NW
CHUNK = 80
NCHUNK = EPW // CHUNK
ZITER = ROWS_PER_TILE // CHUNK

_mesh = plsc.VectorSubcoreMesh(core_axis_name="c", subcore_axis_name="s")


def _sc_degree(dst):
    @functools.partial(
        pl.kernel,
        out_type=jax.ShapeDtypeStruct((NC, NPAD, F_HID), jnp.float32),
        mesh=_mesh,
        scratch_types=[
            pltpu.VMEM((CHUNK,), jnp.int32),
            pltpu.VMEM((CHUNK, F_HID), jnp.float32),
            pltpu.VMEM_SHARED((NPAD, F_HID), jnp.float32),
        ],
    )
    def k(dst_hbm, out_hbm, idx_d, rows, acc):
        cid = lax.axis_index("c")
        sid = lax.axis_index("s")
        wid = sid * NC + cid
        row0 = sid * ROWS_PER_TILE

        zero = jnp.zeros((F_HID,), jnp.float32)

        @pl.loop(0, CHUNK)
        def _(i):
            rows[i, :] = zero

        @pl.loop(0, ZITER)
        def _(j):
            pltpu.sync_copy(rows, acc.at[pl.ds(row0 + j * CHUNK, CHUNK)])

        one = jnp.ones((F_HID,), jnp.float32)

        @pl.loop(0, CHUNK)
        def _(i):
            rows[i, :] = one

        plsc.subcore_barrier()

        base = wid * EPW

        @pl.loop(0, NCHUNK)
        def _(it):
            pltpu.sync_copy(dst_hbm.at[pl.ds(base + it * CHUNK, CHUNK)], idx_d)
            pltpu.sync_copy(rows, acc.at[idx_d], add=True)

        plsc.subcore_barrier()
        pltpu.sync_copy(acc.at[pl.ds(row0, ROWS_PER_TILE)],
                        out_hbm.at[cid, pl.ds(row0, ROWS_PER_TILE)])

    return k(dst)


def _sc_scatter(src, dst, hs):
    @functools.partial(
        pl.kernel,
        out_type=jax.ShapeDtypeStruct((NC, NPAD, F_HID), jnp.float32),
        mesh=_mesh,
        scratch_types=[
            pltpu.VMEM((CHUNK,), jnp.int32),
            pltpu.VMEM((CHUNK,), jnp.int32),
            pltpu.VMEM((CHUNK, F_HID), jnp.float32),
            pltpu.VMEM_SHARED((NPAD, F_HID), jnp.float32),
        ],
    )
    def k(src_hbm, dst_hbm, hs_hbm, out_hbm, idx_s, idx_d, rows, acc):
        cid = lax.axis_index("c")
        sid = lax.axis_index("s")
        wid = sid * NC + cid
        row0 = sid * ROWS_PER_TILE

        zero = jnp.zeros((F_HID,), jnp.float32)

        @pl.loop(0, CHUNK)
        def _(i):
            rows[i, :] = zero

        @pl.loop(0, ZITER)
        def _(j):
            pltpu.sync_copy(rows, acc.at[pl.ds(row0 + j * CHUNK, CHUNK)])

        plsc.subcore_barrier()

        base = wid * EPW

        @pl.loop(0, NCHUNK)
        def _(it):
            off = base + it * CHUNK
            pltpu.sync_copy(src_hbm.at[pl.ds(off, CHUNK)], idx_s)
            pltpu.sync_copy(dst_hbm.at[pl.ds(off, CHUNK)], idx_d)
            pltpu.sync_copy(hs_hbm.at[idx_s], rows)
            pltpu.sync_copy(rows, acc.at[idx_d], add=True)

        plsc.subcore_barrier()
        pltpu.sync_copy(acc.at[pl.ds(row0, ROWS_PER_TILE)],
                        out_hbm.at[cid, pl.ds(row0, ROWS_PER_TILE)])

    return k(src, dst, hs)


_BN = 2000


def _tc_matmul(x, w):
    def body(x_ref, w_ref, o_ref):
        o_ref[...] = jnp.dot(x_ref[...], w_ref[...],
                             preferred_element_type=jnp.float32)

    return pl.pallas_call(
        body,
        grid=(N_NODES // _BN,),
        in_specs=[pl.BlockSpec((_BN, F_IN), lambda i: (i, 0)),
                  pl.BlockSpec((F_IN, F_HID), lambda i: (0, 0))],
        out_specs=pl.BlockSpec((_BN, F_HID), lambda i: (i, 0)),
        out_shape=jax.ShapeDtypeStruct((N_NODES, F_HID), jnp.float32),
    )(x, w)


def _tc_scale(h1, degp):
    def body(h_ref, d_ref, hs_ref, dinv_ref):
        deg = d_ref[0] + d_ref[1] + 1.0
        dinv = lax.rsqrt(jnp.maximum(deg, 1.0))
        dinv_ref[...] = dinv
        hs_ref[...] = dinv * h_ref[...]

    return pl.pallas_call(
        body,
        grid=(N_NODES // _BN,),
        in_specs=[pl.BlockSpec((_BN, F_HID), lambda i: (i, 0)),
                  pl.BlockSpec((2, _BN, F_HID), lambda i: (0, i, 0))],
        out_specs=[pl.BlockSpec((_BN, F_HID), lambda i: (i, 0))] * 2,
        out_shape=[jax.ShapeDtypeStruct((N_NODES, F_HID), jnp.float32)] * 2,
    )(h1, degp)


def _tc_layer2(msgp, hs1, dinv, b1, w2):
    def body(m_ref, hs_ref, dinv_ref, b_ref, w_ref, o_ref):
        agg = m_ref[0] + m_ref[1] + hs_ref[...]
        o1 = jnp.maximum(dinv_ref[...] * agg + b_ref[...], 0.0)
        h2 = jnp.dot(o1, w_ref[...], preferred_element_type=jnp.float32)
        o_ref[...] = dinv_ref[...] * h2

    return pl.pallas_call(
        body,
        grid=(N_NODES // _BN,),
        in_specs=[pl.BlockSpec((2, _BN, F_HID), lambda i: (0, i, 0)),
                  pl.BlockSpec((_BN, F_HID), lambda i: (i, 0)),
                  pl.BlockSpec((_BN, F_HID), lambda i: (i, 0)),
                  pl.BlockSpec((1, F_HID), lambda i: (0, 0)),
                  pl.BlockSpec((F_HID, F_HID), lambda i: (0, 0))],
        out_specs=pl.BlockSpec((_BN, F_HID), lambda i: (i, 0)),
        out_shape=jax.ShapeDtypeStruct((N_NODES, F_HID), jnp.float32),
    )(msgp, hs1, dinv, b1, w2)


def _tc_out(msgp, hs2, dinv, b2):
    def body(m_ref, hs_ref, dinv_ref, b_ref, o_ref):
        agg = m_ref[0] + m_ref[1] + hs_ref[...]
        o2 = dinv_ref[...] * agg + b_ref[...]
        m = jnp.max(o2, axis=1, keepdims=True)
        lse = jnp.log(jnp.sum(jnp.exp(o2 - m), axis=1, keepdims=True))
        o_ref[...] = o2 - m - lse

    return pl.pallas_call(
        body,
        grid=(N_NODES // _BN,),
        in_specs=[pl.BlockSpec((2, _BN, F_HID), lambda i: (0, i, 0)),
                  pl.BlockSpec((_BN, F_HID), lambda i: (i, 0)),
                  pl.BlockSpec((_BN, F_HID), lambda i: (i, 0)),
                  pl.BlockSpec((1, F_HID), lambda i: (0, 0))],
        out_specs=pl.BlockSpec((_BN, F_HID), lambda i: (i, 0)),
        out_shape=jax.ShapeDtypeStruct((N_NODES, F_HID), jnp.float32),
    )(msgp, hs2, dinv, b2)


def kernel(x, edge_index, W1, b1, W2, b2):
    src = edge_index[0]
    dst = edge_index[1]

    degp = _sc_degree(dst)
    h1 = _tc_matmul(x, W1)
    hs1, dinv = _tc_scale(h1, degp[:, :N_NODES, :])
    msg1 = _sc_scatter(src, dst, hs1)
    hs2 = _tc_layer2(msg1[:, :N_NODES, :], hs1, dinv,
                     b1.reshape(1, F_HID), W2)
    msg2 = _sc_scatter(src, dst, hs2)
    return _tc_out(msg2[:, :N_NODES, :], hs2, dinv, b2.reshape(1, F_HID))


# speedup vs baseline: 15.9333x; 15.9333x over previous
import functools

import jax
import jax.numpy as jnp
from jax import lax
from jax.experimental import pallas as pl
from jax.experimental.pallas import tpu as pltpu
from jax.experimental.pallas import tpu_sc as plsc

N_NODES = 10000
N_EDGES = 320000
F_IN = 128
F_HID = 16

NC = 2
NS = 16
NW = NC * NS
NPAD = 10240
ROWS_PER_TILE = NPAD // NS
EPW = N_EDGES // NW
CHUNK = 80
NCHUNK = EPW // CHUNK
ZITER = ROWS_PER_TILE // CHUNK

_mesh = plsc.VectorSubcoreMesh(core_axis_name="c", subcore_axis_name="s")
_sc_params = pltpu.CompilerParams(use_tc_tiling_on_sc=False)


def _sc_degree(dst):
    @functools.partial(
        pl.kernel,
        out_type=jax.ShapeDtypeStruct((NC, NPAD, F_HID), jnp.float32),
        mesh=_mesh,
        compiler_params=_sc_params,
        scratch_types=[
            pltpu.VMEM((CHUNK,), jnp.int32),
            pltpu.VMEM((CHUNK, F_HID), jnp.float32),
            pltpu.VMEM_SHARED((NPAD, F_HID), jnp.float32),
        ],
    )
    def k(dst_hbm, out_hbm, idx_d, rows, acc):
        cid = lax.axis_index("c")
        sid = lax.axis_index("s")
        wid = sid * NC + cid
        row0 = sid * ROWS_PER_TILE

        zero = jnp.zeros((F_HID,), jnp.float32)

        @pl.loop(0, CHUNK)
        def _(i):
            rows[i, :] = zero

        @pl.loop(0, ZITER)
        def _(j):
            pltpu.sync_copy(rows, acc.at[pl.ds(row0 + j * CHUNK, CHUNK)])

        one = jnp.ones((F_HID,), jnp.float32)

        @pl.loop(0, CHUNK)
        def _(i):
            rows[i, :] = one

        plsc.subcore_barrier()

        base = wid * EPW

        @pl.loop(0, NCHUNK)
        def _(it):
            pltpu.sync_copy(dst_hbm.at[pl.ds(base + it * CHUNK, CHUNK)], idx_d)
            pltpu.sync_copy(rows, acc.at[idx_d], add=True)

        plsc.subcore_barrier()
        pltpu.sync_copy(acc.at[pl.ds(row0, ROWS_PER_TILE)],
                        out_hbm.at[cid, pl.ds(row0, ROWS_PER_TILE)])

    return k(dst)


def _sc_scatter(src, dst, hs):
    @functools.partial(
        pl.kernel,
        out_type=jax.ShapeDtypeStruct((NC, NPAD, F_HID), jnp.float32),
        mesh=_mesh,
        compiler_params=_sc_params,
        scratch_types=[
            pltpu.VMEM((CHUNK,), jnp.int32),
            pltpu.VMEM((CHUNK,), jnp.int32),
            pltpu.VMEM((CHUNK, F_HID), jnp.float32),
            pltpu.VMEM_SHARED((NPAD, F_HID), jnp.float32),
        ],
    )
    def k(src_hbm, dst_hbm, hs_hbm, out_hbm, idx_s, idx_d, rows, acc):
        cid = lax.axis_index("c")
        sid = lax.axis_index("s")
        wid = sid * NC + cid
        row0 = sid * ROWS_PER_TILE

        zero = jnp.zeros((F_HID,), jnp.float32)

        @pl.loop(0, CHUNK)
        def _(i):
            rows[i, :] = zero

        @pl.loop(0, ZITER)
        def _(j):
            pltpu.sync_copy(rows, acc.at[pl.ds(row0 + j * CHUNK, CHUNK)])

        plsc.subcore_barrier()

        base = wid * EPW

        @pl.loop(0, NCHUNK)
        def _(it):
            off = base + it * CHUNK
            pltpu.sync_copy(src_hbm.at[pl.ds(off, CHUNK)], idx_s)
            pltpu.sync_copy(dst_hbm.at[pl.ds(off, CHUNK)], idx_d)
            pltpu.sync_copy(hs_hbm.at[idx_s], rows)
            pltpu.sync_copy(rows, acc.at[idx_d], add=True)

        plsc.subcore_barrier()
        pltpu.sync_copy(acc.at[pl.ds(row0, ROWS_PER_TILE)],
                        out_hbm.at[cid, pl.ds(row0, ROWS_PER_TILE)])

    return k(src, dst, hs)


_BN = 2000


def _tc_matmul(x, w):
    def body(x_ref, w_ref, o_ref):
        o_ref[...] = jnp.dot(x_ref[...], w_ref[...],
                             preferred_element_type=jnp.float32)

    return pl.pallas_call(
        body,
        grid=(N_NODES // _BN,),
        in_specs=[pl.BlockSpec((_BN, F_IN), lambda i: (i, 0)),
                  pl.BlockSpec((F_IN, F_HID), lambda i: (0, 0))],
        out_specs=pl.BlockSpec((_BN, F_HID), lambda i: (i, 0)),
        out_shape=jax.ShapeDtypeStruct((N_NODES, F_HID), jnp.float32),
    )(x, w)


def _tc_scale(h1, degp):
    def body(h_ref, d_ref, hs_ref, dinv_ref):
        deg = d_ref[0] + d_ref[1] + 1.0
        dinv = lax.rsqrt(jnp.maximum(deg, 1.0))
        dinv_ref[...] = dinv
        hs_ref[...] = dinv * h_ref[...]

    return pl.pallas_call(
        body,
        grid=(N_NODES // _BN,),
        in_specs=[pl.BlockSpec((_BN, F_HID), lambda i: (i, 0)),
                  pl.BlockSpec((2, _BN, F_HID), lambda i: (0, i, 0))],
        out_specs=[pl.BlockSpec((_BN, F_HID), lambda i: (i, 0))] * 2,
        out_shape=[jax.ShapeDtypeStruct((N_NODES, F_HID), jnp.float32)] * 2,
    )(h1, degp)


def _tc_layer2(msgp, hs1, dinv, b1, w2):
    def body(m_ref, hs_ref, dinv_ref, b_ref, w_ref, o_ref):
        agg = m_ref[0] + m_ref[1] + hs_ref[...]
        o1 = jnp.maximum(dinv_ref[...] * agg + b_ref[...], 0.0)
        h2 = jnp.dot(o1, w_ref[...], preferred_element_type=jnp.float32)
        o_ref[...] = dinv_ref[...] * h2

    return pl.pallas_call(
        body,
        grid=(N_NODES // _BN,),
        in_specs=[pl.BlockSpec((2, _BN, F_HID), lambda i: (0, i, 0)),
                  pl.BlockSpec((_BN, F_HID), lambda i: (i, 0)),
                  pl.BlockSpec((_BN, F_HID), lambda i: (i, 0)),
                  pl.BlockSpec((1, F_HID), lambda i: (0, 0)),
                  pl.BlockSpec((F_HID, F_HID), lambda i: (0, 0))],
        out_specs=pl.BlockSpec((_BN, F_HID), lambda i: (i, 0)),
        out_shape=jax.ShapeDtypeStruct((N_NODES, F_HID), jnp.float32),
    )(msgp, hs1, dinv, b1, w2)


def _tc_out(msgp, hs2, dinv, b2):
    def body(m_ref, hs_ref, dinv_ref, b_ref, o_ref):
        agg = m_ref[0] + m_ref[1] + hs_ref[...]
        o2 = dinv_ref[...] * agg + b_ref[...]
        m = jnp.max(o2, axis=1, keepdims=True)
        lse = jnp.log(jnp.sum(jnp.exp(o2 - m), axis=1, keepdims=True))
        o_ref[...] = o2 - m - lse

    return pl.pallas_call(
        body,
        grid=(N_NODES // _BN,),
        in_specs=[pl.BlockSpec((2, _BN, F_HID), lambda i: (0, i, 0)),
                  pl.BlockSpec((_BN, F_HID), lambda i: (i, 0)),
                  pl.BlockSpec((_BN, F_HID), lambda i: (i, 0)),
                  pl.BlockSpec((1, F_HID), lambda i: (0, 0))],
        out_specs=pl.BlockSpec((_BN, F_HID), lambda i: (i, 0)),
        out_shape=jax.ShapeDtypeStruct((N_NODES, F_HID), jnp.float32),
    )(msgp, hs2, dinv, b2)


def kernel(x, edge_index, W1, b1, W2, b2):
    src = edge_index[0]
    dst = edge_index[1]

    degp = _sc_degree(dst)
    h1 = _tc_matmul(x, W1)
    hs1, dinv = _tc_scale(h1, degp[:, :N_NODES, :])
    msg1 = _sc_scatter(src, dst, hs1)
    hs2 = _tc_layer2(msg1[:, :N_NODES, :], hs1, dinv,
                     b1.reshape(1, F_HID), W2)
    msg2 = _sc_scatter(src, dst, hs2)
    return _tc_out(msg2[:, :N_NODES, :], hs2, dinv, b2.reshape(1, F_HID))


# speedup vs baseline: 52.5625x; 3.2989x over previous
import functools

import jax
import jax.numpy as jnp
from jax import lax
from jax.experimental import pallas as pl
from jax.experimental.pallas import tpu as pltpu
from jax.experimental.pallas import tpu_sc as plsc

N_NODES = 10000
N_EDGES = 320000
F_IN = 128
F_HID = 16

NC = 2
NS = 16
NW = NC * NS
NPAD = 10240
ROWS_PER_TILE = NPAD // NS
EPW = N_EDGES // NW
CHUNK = 125
NCHUNK = EPW // CHUNK
NB = 8
PD = 4
ZROWS = 64
ZITER = ROWS_PER_TILE // ZROWS

_mesh = plsc.VectorSubcoreMesh(core_axis_name="c", subcore_axis_name="s")
_sc_params = pltpu.CompilerParams(use_tc_tiling_on_sc=False)


def _zero_acc_slice(zbuf, acc, row0):
    zero = jnp.zeros((F_HID,), jnp.float32)

    @pl.loop(0, ZROWS)
    def _(i):
        zbuf[i, :] = zero

    @pl.loop(0, ZITER)
    def _(j):
        pltpu.sync_copy(zbuf, acc.at[pl.ds(row0 + j * ZROWS, ZROWS)])


def _sc_degree(dst3):
    @functools.partial(
        pl.kernel,
        out_type=jax.ShapeDtypeStruct((NC, NPAD, F_HID), jnp.float32),
        mesh=_mesh,
        compiler_params=_sc_params,
        scratch_types=[
            pltpu.VMEM((NCHUNK, CHUNK), jnp.int32),
            pltpu.VMEM((CHUNK, F_HID), jnp.float32),
            pltpu.VMEM((ZROWS, F_HID), jnp.float32),
            pltpu.VMEM_SHARED((NPAD, F_HID), jnp.float32),
        ] + [pltpu.SemaphoreType.DMA] * PD,
    )
    def k(dst_hbm, out_hbm, idx_d, ones, zbuf, acc, *ssems):
        cid = lax.axis_index("c")
        sid = lax.axis_index("s")
        wid = sid * NC + cid
        row0 = sid * ROWS_PER_TILE

        pltpu.sync_copy(dst_hbm.at[wid], idx_d)

        one = jnp.ones((F_HID,), jnp.float32)

        @pl.loop(0, CHUNK)
        def _(i):
            ones[i, :] = one

        _zero_acc_slice(zbuf, acc, row0)
        plsc.subcore_barrier()

        @pl.loop(0, NCHUNK, step=PD)
        def _(jo):
            for b in range(PD):
                j = jo + b

                @pl.when(j >= PD)
                def _():
                    pltpu.make_async_copy(
                        ones, acc.at[idx_d.at[j - PD]], ssems[b]).wait()

                pltpu.async_copy(ones, acc.at[idx_d.at[j]], ssems[b],
                                 add=True)

        for b in range(PD):
            j = NCHUNK - PD + b
            pltpu.make_async_copy(ones, acc.at[idx_d.at[j]], ssems[b]).wait()

        plsc.subcore_barrier()
        pltpu.sync_copy(acc.at[pl.ds(row0, ROWS_PER_TILE)],
                        out_hbm.at[cid, pl.ds(row0, ROWS_PER_TILE)])

    return k(dst3)


def _sc_scatter(src3, dst3, hs):
    @functools.partial(
        pl.kernel,
        out_type=jax.ShapeDtypeStruct((NC, NPAD, F_HID), jnp.float32),
        mesh=_mesh,
        compiler_params=_sc_params,
        scratch_types=[
            pltpu.VMEM((NCHUNK, CHUNK), jnp.int32),
            pltpu.VMEM((NCHUNK, CHUNK), jnp.int32),
            pltpu.VMEM((NB, CHUNK, F_HID), jnp.float32),
            pltpu.VMEM((ZROWS, F_HID), jnp.float32),
            pltpu.VMEM_SHARED((NPAD, F_HID), jnp.float32),
        ] + [pltpu.SemaphoreType.DMA] * (2 * NB),
    )
    def k(src_hbm, dst_hbm, hs_hbm, out_hbm, idx_s, idx_d, rows, zbuf, acc,
          *sems):
        gsems, ssems = sems[:NB], sems[NB:]
        cid = lax.axis_index("c")
        sid = lax.axis_index("s")
        wid = sid * NC + cid
        row0 = sid * ROWS_PER_TILE

        pltpu.sync_copy(src_hbm.at[wid], idx_s)
        pltpu.sync_copy(dst_hbm.at[wid], idx_d)
        _zero_acc_slice(zbuf, acc, row0)
        plsc.subcore_barrier()

        for b in range(PD):
            pltpu.async_copy(hs_hbm.at[idx_s.at[b]], rows.at[b], gsems[b])

        @pl.loop(0, NCHUNK, step=NB)
        def _(jo):
            for b in range(NB):
                j = jo + b
                bp = (b + PD) % NB
                pltpu.make_async_copy(
                    hs_hbm.at[idx_s.at[j]], rows.at[b], gsems[b]).wait()
                pltpu.async_copy(rows.at[b], acc.at[idx_d.at[j]], ssems[b],
                                 add=True)
                @pl.when(j >= PD)
                def _():
                    pltpu.make_async_copy(
                        rows.at[bp], acc.at[idx_d.at[j - PD]],
                        ssems[bp]).wait()

                @pl.when(j + PD < NCHUNK)
                def _():
                    pltpu.async_copy(hs_hbm.at[idx_s.at[j + PD]],
                                     rows.at[bp], gsems[bp])

        for b in range(PD):
            j = NCHUNK - PD + b
            pltpu.make_async_copy(rows.at[(j % NB)], acc.at[idx_d.at[j]],
                                  ssems[j % NB]).wait()

        plsc.subcore_barrier()
        pltpu.sync_copy(acc.at[pl.ds(row0, ROWS_PER_TILE)],
                        out_hbm.at[cid, pl.ds(row0, ROWS_PER_TILE)])

    return k(src3, dst3, hs)


_BN = 2000


def _tc_matmul(x, w):
    def body(x_ref, w_ref, o_ref):
        o_ref[...] = jnp.dot(x_ref[...], w_ref[...],
                             preferred_element_type=jnp.float32)

    return pl.pallas_call(
        body,
        grid=(N_NODES // _BN,),
        in_specs=[pl.BlockSpec((_BN, F_IN), lambda i: (i, 0)),
                  pl.BlockSpec((F_IN, F_HID), lambda i: (0, 0))],
        out_specs=pl.BlockSpec((_BN, F_HID), lambda i: (i, 0)),
        out_shape=jax.ShapeDtypeStruct((N_NODES, F_HID), jnp.float32),
    )(x, w)


def _tc_scale(h1, degp):
    def body(h_ref, d_ref, hs_ref, dinv_ref):
        deg = d_ref[0] + d_ref[1] + 1.0
        dinv = lax.rsqrt(jnp.maximum(deg, 1.0))
        dinv_ref[...] = dinv
        hs_ref[...] = dinv * h_ref[...]

    return pl.pallas_call(
        body,
        grid=(N_NODES // _BN,),
        in_specs=[pl.BlockSpec((_BN, F_HID), lambda i: (i, 0)),
                  pl.BlockSpec((2, _BN, F_HID), lambda i: (0, i, 0))],
        out_specs=[pl.BlockSpec((_BN, F_HID), lambda i: (i, 0))] * 2,
        out_shape=[jax.ShapeDtypeStruct((N_NODES, F_HID), jnp.float32)] * 2,
    )(h1, degp)


def _tc_layer2(msgp, hs1, dinv, b1, w2):
    def body(m_ref, hs_ref, dinv_ref, b_ref, w_ref, o_ref):
        agg = m_ref[0] + m_ref[1] + hs_ref[...]
        o1 = jnp.maximum(dinv_ref[...] * agg + b_ref[...], 0.0)
        h2 = jnp.dot(o1, w_ref[...], preferred_element_type=jnp.float32)
        o_ref[...] = dinv_ref[...] * h2

    return pl.pallas_call(
        body,
        grid=(N_NODES // _BN,),
        in_specs=[pl.BlockSpec((2, _BN, F_HID), lambda i: (0, i, 0)),
                  pl.BlockSpec((_BN, F_HID), lambda i: (i, 0)),
                  pl.BlockSpec((_BN, F_HID), lambda i: (i, 0)),
                  pl.BlockSpec((1, F_HID), lambda i: (0, 0)),
                  pl.BlockSpec((F_HID, F_HID), lambda i: (0, 0))],
        out_specs=pl.BlockSpec((_BN, F_HID), lambda i: (i, 0)),
        out_shape=jax.ShapeDtypeStruct((N_NODES, F_HID), jnp.float32),
    )(msgp, hs1, dinv, b1, w2)


def _tc_out(msgp, hs2, dinv, b2):
    def body(m_ref, hs_ref, dinv_ref, b_ref, o_ref):
        agg = m_ref[0] + m_ref[1] + hs_ref[...]
        o2 = dinv_ref[...] * agg + b_ref[...]
        m = jnp.max(o2, axis=1, keepdims=True)
        lse = jnp.log(jnp.sum(jnp.exp(o2 - m), axis=1, keepdims=True))
        o_ref[...] = o2 - m - lse

    return pl.pallas_call(
        body,
        grid=(N_NODES // _BN,),
        in_specs=[pl.BlockSpec((2, _BN, F_HID), lambda i: (0, i, 0)),
                  pl.BlockSpec((_BN, F_HID), lambda i: (i, 0)),
                  pl.BlockSpec((_BN, F_HID), lambda i: (i, 0)),
                  pl.BlockSpec((1, F_HID), lambda i: (0, 0))],
        out_specs=pl.BlockSpec((_BN, F_HID), lambda i: (i, 0)),
        out_shape=jax.ShapeDtypeStruct((N_NODES, F_HID), jnp.float32),
    )(msgp, hs2, dinv, b2)


def kernel(x, edge_index, W1, b1, W2, b2):
    src3 = edge_index[0].reshape(NW, NCHUNK, CHUNK)
    dst3 = edge_index[1].reshape(NW, NCHUNK, CHUNK)

    degp = _sc_degree(dst3)
    h1 = _tc_matmul(x, W1)
    hs1, dinv = _tc_scale(h1, degp[:, :N_NODES, :])
    msg1 = _sc_scatter(src3, dst3, hs1)
    hs2 = _tc_layer2(msg1[:, :N_NODES, :], hs1, dinv,
                     b1.reshape(1, F_HID), W2)
    msg2 = _sc_scatter(src3, dst3, hs2)
    return _tc_out(msg2[:, :N_NODES, :], hs2, dinv, b2.reshape(1, F_HID))


# speedup vs baseline: 59.2050x; 1.1264x over previous
import functools

import jax
import jax.numpy as jnp
from jax import lax
from jax.experimental import pallas as pl
from jax.experimental.pallas import tpu as pltpu
from jax.experimental.pallas import tpu_sc as plsc

N_NODES = 10000
N_EDGES = 320000
F_IN = 128
F_HID = 16

NC = 2
NS = 16
NW = NC * NS
NPAD = 10240
ROWS_PER_TILE = NPAD // NS
EPW = N_EDGES // NW
CHUNK = 125
NCHUNK = EPW // CHUNK
NB = 8
PD = 4
ZROWS = 64
ZITER = ROWS_PER_TILE // ZROWS

_mesh = plsc.VectorSubcoreMesh(core_axis_name="c", subcore_axis_name="s")
_sc_params = pltpu.CompilerParams(use_tc_tiling_on_sc=False)


def _zero_acc_slice(zbuf, acc, row0):
    zero = jnp.zeros((F_HID,), jnp.float32)

    @pl.loop(0, ZROWS)
    def _(i):
        zbuf[i, :] = zero

    @pl.loop(0, ZITER)
    def _(j):
        pltpu.sync_copy(zbuf, acc.at[pl.ds(row0 + j * ZROWS, ZROWS)])


def _sc_degree(dst3):
    @functools.partial(
        pl.kernel,
        out_type=jax.ShapeDtypeStruct((NC, NPAD, F_HID), jnp.float32),
        mesh=_mesh,
        compiler_params=_sc_params,
        scratch_types=[
            pltpu.VMEM((NCHUNK, CHUNK), jnp.int32),
            pltpu.VMEM((CHUNK, F_HID), jnp.float32),
            pltpu.VMEM((ZROWS, F_HID), jnp.float32),
            pltpu.VMEM_SHARED((NPAD, F_HID), jnp.float32),
        ] + [pltpu.SemaphoreType.DMA] * PD,
    )
    def k(dst_hbm, out_hbm, idx_d, ones, zbuf, acc, *ssems):
        cid = lax.axis_index("c")
        sid = lax.axis_index("s")
        wid = sid * NC + cid
        row0 = sid * ROWS_PER_TILE

        pltpu.sync_copy(dst_hbm.at[wid], idx_d)

        one = jnp.ones((F_HID,), jnp.float32)

        @pl.loop(0, CHUNK)
        def _(i):
            ones[i, :] = one

        _zero_acc_slice(zbuf, acc, row0)
        plsc.subcore_barrier()

        @pl.loop(0, NCHUNK, step=PD)
        def _(jo):
            for b in range(PD):
                j = jo + b

                @pl.when(j >= PD)
                def _():
                    pltpu.make_async_copy(
                        ones, acc.at[idx_d.at[j - PD]], ssems[b]).wait()

                pltpu.async_copy(ones, acc.at[idx_d.at[j]], ssems[b],
                                 add=True)

        for b in range(PD):
            j = NCHUNK - PD + b
            pltpu.make_async_copy(ones, acc.at[idx_d.at[j]], ssems[b]).wait()

        plsc.subcore_barrier()
        pltpu.sync_copy(acc.at[pl.ds(row0, ROWS_PER_TILE)],
                        out_hbm.at[cid, pl.ds(row0, ROWS_PER_TILE)])

    return k(dst3)


def _sc_scatter(src3, dst3, hs):
    @functools.partial(
        pl.kernel,
        out_type=jax.ShapeDtypeStruct((NC, NPAD, F_HID), jnp.float32),
        mesh=_mesh,
        compiler_params=_sc_params,
        scratch_types=[
            pltpu.VMEM((NCHUNK, CHUNK), jnp.int32),
            pltpu.VMEM((NCHUNK, CHUNK), jnp.int32),
            pltpu.VMEM((NB, CHUNK, F_HID), jnp.float32),
            pltpu.VMEM((ZROWS, F_HID), jnp.float32),
            pltpu.VMEM_SHARED((NPAD, F_HID), jnp.float32),
        ] + [pltpu.SemaphoreType.DMA] * (2 * NB),
    )
    def k(src_hbm, dst_hbm, hs_hbm, out_hbm, idx_s, idx_d, rows, zbuf, acc,
          *sems):
        gsems, ssems = sems[:NB], sems[NB:]
        cid = lax.axis_index("c")
        sid = lax.axis_index("s")
        wid = sid * NC + cid
        row0 = sid * ROWS_PER_TILE

        pltpu.sync_copy(src_hbm.at[wid], idx_s)
        pltpu.sync_copy(dst_hbm.at[wid], idx_d)
        _zero_acc_slice(zbuf, acc, row0)
        plsc.subcore_barrier()

        for b in range(PD):
            pltpu.async_copy(hs_hbm.at[idx_s.at[b]], rows.at[b], gsems[b])

        @pl.loop(0, NCHUNK, step=NB)
        def _(jo):
            for b in range(NB):
                j = jo + b
                bp = (b + PD) % NB
                pltpu.make_async_copy(
                    hs_hbm.at[idx_s.at[j]], rows.at[b], gsems[b]).wait()
                pltpu.async_copy(rows.at[b], acc.at[idx_d.at[j]], ssems[b],
                                 add=True)
                @pl.when(j >= PD)
                def _():
                    pltpu.make_async_copy(
                        rows.at[bp], acc.at[idx_d.at[j - PD]],
                        ssems[bp]).wait()

                @pl.when(j + PD < NCHUNK)
                def _():
                    pltpu.async_copy(hs_hbm.at[idx_s.at[j + PD]],
                                     rows.at[bp], gsems[bp])

        for b in range(PD):
            j = NCHUNK - PD + b
            pltpu.make_async_copy(rows.at[(j % NB)], acc.at[idx_d.at[j]],
                                  ssems[j % NB]).wait()

        plsc.subcore_barrier()
        pltpu.sync_copy(acc.at[pl.ds(row0, ROWS_PER_TILE)],
                        out_hbm.at[cid, pl.ds(row0, ROWS_PER_TILE)])

    return k(src3, dst3, hs)


_BN = 2000


def _tc_matmul_scale(x, w, degp):
    def body(x_ref, w_ref, d_ref, hs_ref, dinv_ref):
        h1 = jnp.dot(x_ref[...], w_ref[...],
                     preferred_element_type=jnp.float32)
        deg = d_ref[0] + d_ref[1] + 1.0
        dinv = lax.rsqrt(jnp.maximum(deg, 1.0))
        dinv_ref[...] = dinv
        hs_ref[...] = dinv * h1

    return pl.pallas_call(
        body,
        grid=(N_NODES // _BN,),
        in_specs=[pl.BlockSpec((_BN, F_IN), lambda i: (i, 0)),
                  pl.BlockSpec((F_IN, F_HID), lambda i: (0, 0)),
                  pl.BlockSpec((2, _BN, F_HID), lambda i: (0, i, 0))],
        out_specs=[pl.BlockSpec((_BN, F_HID), lambda i: (i, 0))] * 2,
        out_shape=[jax.ShapeDtypeStruct((N_NODES, F_HID), jnp.float32)] * 2,
    )(x, w, degp)


def _tc_layer2(msgp, hs1, dinv, b1, w2):
    def body(m_ref, hs_ref, dinv_ref, b_ref, w_ref, o_ref):
        agg = m_ref[0] + m_ref[1] + hs_ref[...]
        o1 = jnp.maximum(dinv_ref[...] * agg + b_ref[...], 0.0)
        h2 = jnp.dot(o1, w_ref[...], preferred_element_type=jnp.float32)
        o_ref[...] = dinv_ref[...] * h2

    return pl.pallas_call(
        body,
        grid=(N_NODES // _BN,),
        in_specs=[pl.BlockSpec((2, _BN, F_HID), lambda i: (0, i, 0)),
                  pl.BlockSpec((_BN, F_HID), lambda i: (i, 0)),
                  pl.BlockSpec((_BN, F_HID), lambda i: (i, 0)),
                  pl.BlockSpec((1, F_HID), lambda i: (0, 0)),
                  pl.BlockSpec((F_HID, F_HID), lambda i: (0, 0))],
        out_specs=pl.BlockSpec((_BN, F_HID), lambda i: (i, 0)),
        out_shape=jax.ShapeDtypeStruct((N_NODES, F_HID), jnp.float32),
    )(msgp, hs1, dinv, b1, w2)


def _tc_out(msgp, hs2, dinv, b2):
    def body(m_ref, hs_ref, dinv_ref, b_ref, o_ref):
        agg = m_ref[0] + m_ref[1] + hs_ref[...]
        o2 = dinv_ref[...] * agg + b_ref[...]
        m = jnp.max(o2, axis=1, keepdims=True)
        lse = jnp.log(jnp.sum(jnp.exp(o2 - m), axis=1, keepdims=True))
        o_ref[...] = o2 - m - lse

    return pl.pallas_call(
        body,
        grid=(N_NODES // _BN,),
        in_specs=[pl.BlockSpec((2, _BN, F_HID), lambda i: (0, i, 0)),
                  pl.BlockSpec((_BN, F_HID), lambda i: (i, 0)),
                  pl.BlockSpec((_BN, F_HID), lambda i: (i, 0)),
                  pl.BlockSpec((1, F_HID), lambda i: (0, 0))],
        out_specs=pl.BlockSpec((_BN, F_HID), lambda i: (i, 0)),
        out_shape=jax.ShapeDtypeStruct((N_NODES, F_HID), jnp.float32),
    )(msgp, hs2, dinv, b2)


def kernel(x, edge_index, W1, b1, W2, b2):
    src3 = edge_index[0].reshape(NW, NCHUNK, CHUNK)
    dst3 = edge_index[1].reshape(NW, NCHUNK, CHUNK)

    degp = _sc_degree(dst3)
    hs1, dinv = _tc_matmul_scale(x, W1, degp)
    msg1 = _sc_scatter(src3, dst3, hs1)
    hs2 = _tc_layer2(msg1, hs1, dinv, b1.reshape(1, F_HID), W2)
    msg2 = _sc_scatter(src3, dst3, hs2)
    return _tc_out(msg2, hs2, dinv, b2.reshape(1, F_HID))


# speedup vs baseline: 67.7961x; 1.1451x over previous
import functools

import jax
import jax.numpy as jnp
from jax import lax
from jax.experimental import pallas as pl
from jax.experimental.pallas import tpu as pltpu
from jax.experimental.pallas import tpu_sc as plsc

N_NODES = 10000
N_EDGES = 320000
F_IN = 128
F_HID = 16

NC = 2
NS = 16
NW = NC * NS
NPAD = 10240
ROWS_PER_TILE = NPAD // NS
EPW = N_EDGES // NW
CHUNK = 125
NCHUNK = EPW // CHUNK
NB = 8
PD = 4
ZROWS = 64
ZITER = ROWS_PER_TILE // ZROWS

_mesh = plsc.VectorSubcoreMesh(core_axis_name="c", subcore_axis_name="s")
_sc_params = pltpu.CompilerParams(use_tc_tiling_on_sc=False)


def _zero_acc_slice(zbuf, acc, row0):
    zero = jnp.zeros((F_HID,), jnp.float32)

    @pl.loop(0, ZROWS)
    def _(i):
        zbuf[i, :] = zero

    @pl.loop(0, ZITER)
    def _(j):
        pltpu.sync_copy(zbuf, acc.at[pl.ds(row0 + j * ZROWS, ZROWS)])


def _sc_degree(edge3):
    @functools.partial(
        pl.kernel,
        out_type=jax.ShapeDtypeStruct((NC, NPAD), jnp.float32),
        mesh=_mesh,
        compiler_params=_sc_params,
        scratch_types=[
            pltpu.VMEM((NCHUNK, CHUNK), jnp.int32),
            pltpu.VMEM((128,), jnp.float32),
            pltpu.VMEM((ROWS_PER_TILE,), jnp.float32),
            pltpu.VMEM_SHARED((NPAD,), jnp.float32),
        ] + [pltpu.SemaphoreType.DMA] * PD,
    )
    def k(e_hbm, out_hbm, idx_d, ones128, zbuf, acc, *ssems):
        ones = ones128.at[pl.ds(0, CHUNK)]
        cid = lax.axis_index("c")
        sid = lax.axis_index("s")
        wid = sid * NC + cid
        row0 = sid * ROWS_PER_TILE

        pltpu.sync_copy(e_hbm.at[1, wid], idx_d)

        one = jnp.ones((16,), jnp.float32)

        @pl.loop(0, 8)
        def _(i):
            ones128[pl.ds(i * 16, 16)] = one

        @pl.loop(0, ROWS_PER_TILE // 16)
        def _(i):
            zbuf[pl.ds(i * 16, 16)] = jnp.zeros((16,), jnp.float32)

        pltpu.sync_copy(zbuf, acc.at[pl.ds(row0, ROWS_PER_TILE)])
        plsc.subcore_barrier()

        @pl.loop(0, NCHUNK, step=PD)
        def _(jo):
            for b in range(PD):
                j = jo + b

                @pl.when(j >= PD)
                def _():
                    pltpu.make_async_copy(
                        ones, acc.at[idx_d.at[j - PD]], ssems[b]).wait()

                pltpu.async_copy(ones, acc.at[idx_d.at[j]], ssems[b],
                                 add=True)

        for b in range(PD):
            j = NCHUNK - PD + b
            pltpu.make_async_copy(ones, acc.at[idx_d.at[j]], ssems[b]).wait()

        plsc.subcore_barrier()
        pltpu.sync_copy(acc.at[pl.ds(row0, ROWS_PER_TILE)],
                        out_hbm.at[cid, pl.ds(row0, ROWS_PER_TILE)])

    return k(edge3)


def _sc_scatter(edge3, hs):
    @functools.partial(
        pl.kernel,
        out_type=jax.ShapeDtypeStruct((NC, NPAD, F_HID), jnp.float32),
        mesh=_mesh,
        compiler_params=_sc_params,
        scratch_types=[
            pltpu.VMEM((NCHUNK, CHUNK), jnp.int32),
            pltpu.VMEM((NCHUNK, CHUNK), jnp.int32),
            pltpu.VMEM((NB, CHUNK, F_HID), jnp.float32),
            pltpu.VMEM((ZROWS, F_HID), jnp.float32),
            pltpu.VMEM_SHARED((NPAD, F_HID), jnp.float32),
        ] + [pltpu.SemaphoreType.DMA] * (2 * NB),
    )
    def k(e_hbm, hs_hbm, out_hbm, idx_s, idx_d, rows, zbuf, acc, *sems):
        gsems, ssems = sems[:NB], sems[NB:]
        cid = lax.axis_index("c")
        sid = lax.axis_index("s")
        wid = sid * NC + cid
        row0 = sid * ROWS_PER_TILE

        pltpu.sync_copy(e_hbm.at[0, wid], idx_s)
        pltpu.sync_copy(e_hbm.at[1, wid], idx_d)
        _zero_acc_slice(zbuf, acc, row0)
        plsc.subcore_barrier()

        for b in range(PD):
            pltpu.async_copy(hs_hbm.at[idx_s.at[b]], rows.at[b], gsems[b])

        @pl.loop(0, NCHUNK, step=NB)
        def _(jo):
            for b in range(NB):
                j = jo + b
                bp = (b + PD) % NB
                pltpu.make_async_copy(
                    hs_hbm.at[idx_s.at[j]], rows.at[b], gsems[b]).wait()
                pltpu.async_copy(rows.at[b], acc.at[idx_d.at[j]], ssems[b],
                                 add=True)
                @pl.when(j >= PD)
                def _():
                    pltpu.make_async_copy(
                        rows.at[bp], acc.at[idx_d.at[j - PD]],
                        ssems[bp]).wait()

                @pl.when(j + PD < NCHUNK)
                def _():
                    pltpu.async_copy(hs_hbm.at[idx_s.at[j + PD]],
                                     rows.at[bp], gsems[bp])

        for b in range(PD):
            j = NCHUNK - PD + b
            pltpu.make_async_copy(rows.at[(j % NB)], acc.at[idx_d.at[j]],
                                  ssems[j % NB]).wait()

        plsc.subcore_barrier()
        pltpu.sync_copy(acc.at[pl.ds(row0, ROWS_PER_TILE)],
                        out_hbm.at[cid, pl.ds(row0, ROWS_PER_TILE)])

    return k(edge3, hs)


_BN = 2000


def _tc_matmul_scale(x, w, degp):
    def body(x_ref, w_ref, d_ref, hs_ref, dinv_ref):
        h1 = jnp.dot(x_ref[...], w_ref[...],
                     preferred_element_type=jnp.float32)
        deg = d_ref[0, pl.ds(0, N_NODES)] + d_ref[1, pl.ds(0, N_NODES)] + 1.0
        dinv = lax.rsqrt(jnp.maximum(deg, 1.0))[:, None]
        dinv_ref[...] = jnp.broadcast_to(dinv, (N_NODES, F_HID))
        hs_ref[...] = dinv * h1

    return pl.pallas_call(
        body,
        out_shape=[jax.ShapeDtypeStruct((N_NODES, F_HID), jnp.float32)] * 2,
    )(x, w, degp)


def _tc_layer2(msgp, hs1, dinv, b1, w2):
    def body(m_ref, hs_ref, dinv_ref, b_ref, w_ref, o_ref):
        blk = pl.ds(0, N_NODES)
        agg = m_ref[0, blk] + m_ref[1, blk] + hs_ref[...]
        o1 = jnp.maximum(dinv_ref[...] * agg + b_ref[...], 0.0)
        h2 = jnp.dot(o1, w_ref[...], preferred_element_type=jnp.float32)
        o_ref[...] = dinv_ref[...] * h2

    return pl.pallas_call(
        body,
        out_shape=jax.ShapeDtypeStruct((N_NODES, F_HID), jnp.float32),
    )(msgp, hs1, dinv, b1, w2)


def _tc_out(msgp, hs2, dinv, b2):
    def body(m_ref, hs_ref, dinv_ref, b_ref, o_ref):
        blk = pl.ds(0, N_NODES)
        agg = m_ref[0, blk] + m_ref[1, blk] + hs_ref[...]
        o2 = dinv_ref[...] * agg + b_ref[...]
        m = jnp.max(o2, axis=1, keepdims=True)
        lse = jnp.log(jnp.sum(jnp.exp(o2 - m), axis=1, keepdims=True))
        o_ref[...] = o2 - m - lse

    return pl.pallas_call(
        body,
        out_shape=jax.ShapeDtypeStruct((N_NODES, F_HID), jnp.float32),
    )(msgp, hs2, dinv, b2)


def kernel(x, edge_index, W1, b1, W2, b2):
    edge3 = edge_index.reshape(2, NW, NCHUNK, CHUNK)

    degp = _sc_degree(edge3)
    hs1, dinv = _tc_matmul_scale(x, W1, degp)
    msg1 = _sc_scatter(edge3, hs1)
    hs2 = _tc_layer2(msg1, hs1, dinv, b1.reshape(1, F_HID), W2)
    msg2 = _sc_scatter(edge3, hs2)
    return _tc_out(msg2, hs2, dinv, b2.reshape(1, F_HID))


# speedup vs baseline: 70.4552x; 1.0392x over previous
import functools

import jax
import jax.numpy as jnp
from jax import lax
from jax.experimental import pallas as pl
from jax.experimental.pallas import tpu as pltpu
from jax.experimental.pallas import tpu_sc as plsc

N_NODES = 10000
N_EDGES = 320000
F_IN = 128
F_HID = 16

NC = 2
NS = 16
NW = NC * NS
NPAD = 10240
ROWS_PER_TILE = NPAD // NS
EPW = N_EDGES // NW
CHUNK = 125
NCHUNK = EPW // CHUNK
NB = 10
PD = 5
ZROWS = 64
ZITER = ROWS_PER_TILE // ZROWS

_mesh = plsc.VectorSubcoreMesh(core_axis_name="c", subcore_axis_name="s")
_sc_params = pltpu.CompilerParams(use_tc_tiling_on_sc=False)


def _zero_acc_slice(zbuf, acc, row0):
    zero = jnp.zeros((F_HID,), jnp.float32)

    @pl.loop(0, ZROWS)
    def _(i):
        zbuf[i, :] = zero

    @pl.loop(0, ZITER)
    def _(j):
        pltpu.sync_copy(zbuf, acc.at[pl.ds(row0 + j * ZROWS, ZROWS)])


def _sc_degree(edge3):
    @functools.partial(
        pl.kernel,
        out_type=jax.ShapeDtypeStruct((NC, NPAD), jnp.float32),
        mesh=_mesh,
        compiler_params=_sc_params,
        scratch_types=[
            pltpu.VMEM((NCHUNK, CHUNK), jnp.int32),
            pltpu.VMEM((128,), jnp.float32),
            pltpu.VMEM((ROWS_PER_TILE,), jnp.float32),
            pltpu.VMEM_SHARED((NPAD,), jnp.float32),
        ] + [pltpu.SemaphoreType.DMA] * PD,
    )
    def k(e_hbm, out_hbm, idx_d, ones128, zbuf, acc, *ssems):
        ones = ones128.at[pl.ds(0, CHUNK)]
        cid = lax.axis_index("c")
        sid = lax.axis_index("s")
        wid = sid * NC + cid
        row0 = sid * ROWS_PER_TILE

        pltpu.sync_copy(e_hbm.at[1, wid], idx_d)

        one = jnp.ones((16,), jnp.float32)

        @pl.loop(0, 8)
        def _(i):
            ones128[pl.ds(i * 16, 16)] = one

        @pl.loop(0, ROWS_PER_TILE // 16)
        def _(i):
            zbuf[pl.ds(i * 16, 16)] = jnp.zeros((16,), jnp.float32)

        pltpu.sync_copy(zbuf, acc.at[pl.ds(row0, ROWS_PER_TILE)])
        plsc.subcore_barrier()

        @pl.loop(0, NCHUNK, step=PD)
        def _(jo):
            for b in range(PD):
                j = jo + b

                @pl.when(j >= PD)
                def _():
                    pltpu.make_async_copy(
                        ones, acc.at[idx_d.at[j - PD]], ssems[b]).wait()

                pltpu.async_copy(ones, acc.at[idx_d.at[j]], ssems[b],
                                 add=True)

        for b in range(PD):
            j = NCHUNK - PD + b
            pltpu.make_async_copy(ones, acc.at[idx_d.at[j]], ssems[b]).wait()

        plsc.subcore_barrier()
        pltpu.sync_copy(acc.at[pl.ds(row0, ROWS_PER_TILE)],
                        out_hbm.at[cid, pl.ds(row0, ROWS_PER_TILE)])

    return k(edge3)


def _sc_scatter(edge3, hs):
    @functools.partial(
        pl.kernel,
        out_type=jax.ShapeDtypeStruct((NC, NPAD, F_HID), jnp.float32),
        mesh=_mesh,
        compiler_params=_sc_params,
        scratch_types=[
            pltpu.VMEM((NCHUNK, CHUNK), jnp.int32),
            pltpu.VMEM((NCHUNK, CHUNK), jnp.int32),
            pltpu.VMEM((NB, CHUNK, F_HID), jnp.float32),
            pltpu.VMEM((ZROWS, F_HID), jnp.float32),
            pltpu.VMEM_SHARED((NPAD, F_HID), jnp.float32),
        ] + [pltpu.SemaphoreType.DMA] * (2 * NB),
    )
    def k(e_hbm, hs_hbm, out_hbm, idx_s, idx_d, rows, zbuf, acc, *sems):
        gsems, ssems = sems[:NB], sems[NB:]
        cid = lax.axis_index("c")
        sid = lax.axis_index("s")
        wid = sid * NC + cid
        row0 = sid * ROWS_PER_TILE

        pltpu.sync_copy(e_hbm.at[0, wid], idx_s)
        pltpu.sync_copy(e_hbm.at[1, wid], idx_d)
        _zero_acc_slice(zbuf, acc, row0)
        plsc.subcore_barrier()

        for b in range(PD):
            pltpu.async_copy(hs_hbm.at[idx_s.at[b]], rows.at[b], gsems[b])

        @pl.loop(0, NCHUNK, step=NB)
        def _(jo):
            for b in range(NB):
                j = jo + b
                bp = (b + PD) % NB
                pltpu.make_async_copy(
                    hs_hbm.at[idx_s.at[j]], rows.at[b], gsems[b]).wait()
                pltpu.async_copy(rows.at[b], acc.at[idx_d.at[j]], ssems[b],
                                 add=True)
                @pl.when(j >= PD)
                def _():
                    pltpu.make_async_copy(
                        rows.at[bp], acc.at[idx_d.at[j - PD]],
                        ssems[bp]).wait()

                @pl.when(j + PD < NCHUNK)
                def _():
                    pltpu.async_copy(hs_hbm.at[idx_s.at[j + PD]],
                                     rows.at[bp], gsems[bp])

        for b in range(PD):
            j = NCHUNK - PD + b
            pltpu.make_async_copy(rows.at[(j % NB)], acc.at[idx_d.at[j]],
                                  ssems[j % NB]).wait()

        plsc.subcore_barrier()
        pltpu.sync_copy(acc.at[pl.ds(row0, ROWS_PER_TILE)],
                        out_hbm.at[cid, pl.ds(row0, ROWS_PER_TILE)])

    return k(edge3, hs)


_BN = 2000


def _tc_matmul_scale(x, w, degp):
    def body(x_ref, w_ref, d_ref, hs_ref, dinv_ref):
        h1 = jnp.dot(x_ref[...], w_ref[...],
                     preferred_element_type=jnp.float32)
        deg = d_ref[0, pl.ds(0, N_NODES)] + d_ref[1, pl.ds(0, N_NODES)] + 1.0
        dinv = lax.rsqrt(jnp.maximum(deg, 1.0))[:, None]
        dinv_ref[...] = jnp.broadcast_to(dinv, (N_NODES, F_HID))
        hs_ref[...] = dinv * h1

    return pl.pallas_call(
        body,
        out_shape=[jax.ShapeDtypeStruct((N_NODES, F_HID), jnp.float32)] * 2,
    )(x, w, degp)


def _tc_layer2(msgp, hs1, dinv, b1, w2):
    def body(m_ref, hs_ref, dinv_ref, b_ref, w_ref, o_ref):
        blk = pl.ds(0, N_NODES)
        agg = m_ref[0, blk] + m_ref[1, blk] + hs_ref[...]
        o1 = jnp.maximum(dinv_ref[...] * agg + b_ref[...], 0.0)
        h2 = jnp.dot(o1, w_ref[...], preferred_element_type=jnp.float32)
        o_ref[...] = dinv_ref[...] * h2

    return pl.pallas_call(
        body,
        out_shape=jax.ShapeDtypeStruct((N_NODES, F_HID), jnp.float32),
    )(msgp, hs1, dinv, b1, w2)


def _tc_out(msgp, hs2, dinv, b2):
    def body(m_ref, hs_ref, dinv_ref, b_ref, o_ref):
        blk = pl.ds(0, N_NODES)
        agg = m_ref[0, blk] + m_ref[1, blk] + hs_ref[...]
        o2 = dinv_ref[...] * agg + b_ref[...]
        m = jnp.max(o2, axis=1, keepdims=True)
        lse = jnp.log(jnp.sum(jnp.exp(o2 - m), axis=1, keepdims=True))
        o_ref[...] = o2 - m - lse

    return pl.pallas_call(
        body,
        out_shape=jax.ShapeDtypeStruct((N_NODES, F_HID), jnp.float32),
    )(msgp, hs2, dinv, b2)


def kernel(x, edge_index, W1, b1, W2, b2):
    edge3 = edge_index.reshape(2, NW, NCHUNK, CHUNK)

    degp = _sc_degree(edge3)
    hs1, dinv = _tc_matmul_scale(x, W1, degp)
    msg1 = _sc_scatter(edge3, hs1)
    hs2 = _tc_layer2(msg1, hs1, dinv, b1.reshape(1, F_HID), W2)
    msg2 = _sc_scatter(edge3, hs2)
    return _tc_out(msg2, hs2, dinv, b2.reshape(1, F_HID))
